# baseline (speedup 1.0000x reference)
import functools

import jax
import jax.numpy as jnp
from jax import lax
from jax.experimental import pallas as pl
from jax.experimental.pallas import tpu as pltpu
from jax.experimental.pallas import tpu_sc as plsc

_LANES = 16
_CHUNK = 32 * 1024


def _make_sc_kernel(B, L, D):
    info = plsc.get_sparse_core_info()
    NC, NS = info.num_cores, info.num_subcores
    NW = NC * NS
    total = B * L * D
    per_w = total // NW
    emb_total = L * D
    n_chunks = per_w // _CHUNK

    mesh = plsc.VectorSubcoreMesh(core_axis_name="c", subcore_axis_name="s")

    @functools.partial(
        pl.kernel,
        mesh=mesh,
        out_type=jax.ShapeDtypeStruct((total,), jnp.float32),
        scratch_types=[
            pltpu.VMEM((_CHUNK,), jnp.float32),
            pltpu.VMEM((_CHUNK,), jnp.float32),
        ],
    )
    def k(xf, ef, of, xb, eb):
        wid = lax.axis_index("s") * NC + lax.axis_index("c")
        xbase = wid * per_w

        def chunk_body(c, _):
            off = xbase + c * _CHUNK
            eoff = lax.rem(off, emb_total)
            pltpu.sync_copy(xf.at[pl.ds(off, _CHUNK)], xb)
            pltpu.sync_copy(ef.at[pl.ds(eoff, _CHUNK)], eb)

            def add_body(j, _):
                s = pl.ds(j * _LANES, _LANES)
                xb[s] = xb[s] + eb[s]
                return 0

            lax.fori_loop(0, _CHUNK // _LANES, add_body, 0)
            pltpu.sync_copy(xb, of.at[pl.ds(off, _CHUNK)])
            return 0

        lax.fori_loop(0, n_chunks, chunk_body, 0)

    return k


def _pos_add_3d(x, emb_slice):
    B, L, D = x.shape
    out = _make_sc_kernel(B, L, D)(
        jnp.reshape(x, (-1,)), jnp.reshape(emb_slice, (-1,))
    )
    return jnp.reshape(out, (B, L, D))


def kernel(x, emb_table):
    if x.ndim == 3:
        L = x.shape[-2]
        return _pos_add_3d(x, emb_table[:L])
    b, h, l, d = x.shape
    xr = jnp.reshape(jnp.transpose(x, (0, 2, 1, 3)), (b, l, h * d))
    xr = _pos_add_3d(xr, emb_table[:l])
    return jnp.transpose(jnp.reshape(xr, (b, l, h, d)), (0, 2, 1, 3))


# speedup vs baseline: 1.5825x; 1.5825x over previous
import functools

import jax
import jax.numpy as jnp
from jax import lax
from jax.experimental import pallas as pl
from jax.experimental.pallas import tpu as pltpu
from jax.experimental.pallas import tpu_sc as plsc

_LANES = 16
_CHUNK = 16 * 1024
_NBUF = 3
_UNROLL = 4


def _make_sc_kernel(B, L, D):
    info = plsc.get_sparse_core_info()
    NC, NS = info.num_cores, info.num_subcores
    NW = NC * NS
    total = B * L * D
    per_w = total // NW
    emb_total = L * D
    n_chunks = per_w // _CHUNK

    mesh = plsc.VectorSubcoreMesh(core_axis_name="c", subcore_axis_name="s")

    @functools.partial(
        pl.kernel,
        mesh=mesh,
        out_type=jax.ShapeDtypeStruct((total,), jnp.float32),
        scratch_types=(
            [pltpu.VMEM((_CHUNK,), jnp.float32)] * (2 * _NBUF)
            + [
                pltpu.SemaphoreType.DMA((_NBUF,)),
                pltpu.SemaphoreType.DMA((_NBUF,)),
                pltpu.SemaphoreType.DMA((_NBUF,)),
            ]
        ),
    )
    def k(xf, ef, of, xb0, xb1, xb2, eb0, eb1, eb2, sem_x, sem_e, sem_o):
        xbs = (xb0, xb1, xb2)
        ebs = (eb0, eb1, eb2)
        wid = lax.axis_index("s") * NC + lax.axis_index("c")
        xbase = wid * per_w

        def xcopy(c):
            slot = c % _NBUF
            off = xbase + c * _CHUNK
            return pltpu.make_async_copy(
                xf.at[pl.ds(off, _CHUNK)], xbs[slot], sem_x.at[slot]
            )

        def ecopy(c):
            slot = c % _NBUF
            eoff = lax.rem(xbase + c * _CHUNK, emb_total)
            return pltpu.make_async_copy(
                ef.at[pl.ds(eoff, _CHUNK)], ebs[slot], sem_e.at[slot]
            )

        def ocopy(c):
            slot = c % _NBUF
            off = xbase + c * _CHUNK
            return pltpu.make_async_copy(
                xbs[slot], of.at[pl.ds(off, _CHUNK)], sem_o.at[slot]
            )

        for c in range(_NBUF - 1):
            xcopy(c).start()
            ecopy(c).start()
        for c in range(n_chunks):
            slot = c % _NBUF
            nxt = c + _NBUF - 1
            if nxt < n_chunks:
                if c >= 1:
                    ocopy(c - 1).wait()
                xcopy(nxt).start()
                ecopy(nxt).start()
            xcopy(c).wait()
            ecopy(c).wait()

            def add_body(j, _, xv=xbs[slot], ev=ebs[slot]):
                base = j * (_LANES * _UNROLL)
                for u in range(_UNROLL):
                    s = pl.ds(base + u * _LANES, _LANES)
                    xv[s] = xv[s] + ev[s]
                return 0

            lax.fori_loop(0, _CHUNK // (_LANES * _UNROLL), add_body, 0)
            ocopy(c).start()
        for c in range(n_chunks - _NBUF, n_chunks):
            ocopy(c).wait()

    return k


def _pos_add_3d(x, emb_slice):
    B, L, D = x.shape
    out = _make_sc_kernel(B, L, D)(
        jnp.reshape(x, (-1,)), jnp.reshape(emb_slice, (-1,))
    )
    return jnp.reshape(out, (B, L, D))


def kernel(x, emb_table):
    if x.ndim == 3:
        L = x.shape[-2]
        return _pos_add_3d(x, emb_table[:L])
    b, h, l, d = x.shape
    xr = jnp.reshape(jnp.transpose(x, (0, 2, 1, 3)), (b, l, h * d))
    xr = _pos_add_3d(xr, emb_table[:l])
    return jnp.transpose(jnp.reshape(xr, (b, l, h, d)), (0, 2, 1, 3))


# speedup vs baseline: 2.1295x; 1.3457x over previous
import functools

import jax
import jax.numpy as jnp
from jax import lax
from jax.experimental import pallas as pl
from jax.experimental.pallas import tpu as pltpu
from jax.experimental.pallas import tpu_sc as plsc

_LANES = 16
_CHUNK = 16 * 1024
_NBUF = 3
_UNROLL = 4
_LSC = 256


def _make_sc_kernel(B, L, D):
    info = plsc.get_sparse_core_info()
    NC, NS = info.num_cores, info.num_subcores
    NW = NC * NS
    total = B * L * D
    per_w = total // NW
    emb_total = L * D
    n_chunks = per_w // _CHUNK

    mesh = plsc.VectorSubcoreMesh(core_axis_name="c", subcore_axis_name="s")

    @functools.partial(
        pl.kernel,
        mesh=mesh,
        out_type=jax.ShapeDtypeStruct((total,), jnp.float32),
        scratch_types=(
            [pltpu.VMEM((_CHUNK,), jnp.float32)] * (2 * _NBUF)
            + [
                pltpu.SemaphoreType.DMA((_NBUF,)),
                pltpu.SemaphoreType.DMA((_NBUF,)),
                pltpu.SemaphoreType.DMA((_NBUF,)),
            ]
        ),
    )
    def k(xf, ef, of, xb0, xb1, xb2, eb0, eb1, eb2, sem_x, sem_e, sem_o):
        xbs = (xb0, xb1, xb2)
        ebs = (eb0, eb1, eb2)
        wid = lax.axis_index("s") * NC + lax.axis_index("c")
        xbase = wid * per_w

        def xcopy(c):
            slot = c % _NBUF
            off = xbase + c * _CHUNK
            return pltpu.make_async_copy(
                xf.at[pl.ds(off, _CHUNK)], xbs[slot], sem_x.at[slot]
            )

        def ecopy(c):
            slot = c % _NBUF
            eoff = lax.rem(xbase + c * _CHUNK, emb_total)
            return pltpu.make_async_copy(
                ef.at[pl.ds(eoff, _CHUNK)], ebs[slot], sem_e.at[slot]
            )

        def ocopy(c):
            slot = c % _NBUF
            off = xbase + c * _CHUNK
            return pltpu.make_async_copy(
                xbs[slot], of.at[pl.ds(off, _CHUNK)], sem_o.at[slot]
            )

        for c in range(min(_NBUF - 1, n_chunks)):
            xcopy(c).start()
            ecopy(c).start()
        for c in range(n_chunks):
            slot = c % _NBUF
            nxt = c + _NBUF - 1
            if nxt < n_chunks:
                if c >= 1:
                    ocopy(c - 1).wait()
                xcopy(nxt).start()
                ecopy(nxt).start()
            xcopy(c).wait()
            ecopy(c).wait()

            def add_body(j, _, xv=xbs[slot], ev=ebs[slot]):
                base = j * (_LANES * _UNROLL)
                for u in range(_UNROLL):
                    s = pl.ds(base + u * _LANES, _LANES)
                    xv[s] = xv[s] + ev[s]
                return 0

            lax.fori_loop(0, _CHUNK // (_LANES * _UNROLL), add_body, 0)
            ocopy(c).start()
        for c in range(max(n_chunks - _NBUF, 0), n_chunks):
            ocopy(c).wait()

    return k


def _tc_add_kernel(x_ref, emb_ref, o_ref):
    o_ref[...] = x_ref[...] + emb_ref[...][None, :, :]


def _tc_pos_add(x, emb_slice):
    B, L, D = x.shape
    BLK = 256
    return pl.pallas_call(
        _tc_add_kernel,
        grid=(L // BLK,),
        in_specs=[
            pl.BlockSpec((B, BLK, D), lambda i: (0, i, 0)),
            pl.BlockSpec((BLK, D), lambda i: (i, 0)),
        ],
        out_specs=pl.BlockSpec((B, BLK, D), lambda i: (0, i, 0)),
        out_shape=jax.ShapeDtypeStruct((B, L, D), x.dtype),
    )(x, emb_slice)


def _sc_pos_add(x, emb_slice):
    B, L, D = x.shape
    out = _make_sc_kernel(B, L, D)(
        jnp.reshape(x, (-1,)), jnp.reshape(emb_slice, (-1,))
    )
    return jnp.reshape(out, (B, L, D))


def _pos_add_3d(x, emb_slice):
    B, L, D = x.shape
    lsc = _LSC
    if L <= lsc or (B * lsc * D) % (32 * _CHUNK) != 0 or (L - lsc) % 256 != 0:
        return _tc_pos_add(x, emb_slice)
    sc_out = _sc_pos_add(x[:, :lsc, :], emb_slice[:lsc])
    tc_out = _tc_pos_add(x[:, lsc:, :], emb_slice[lsc:])
    return jnp.concatenate([sc_out, tc_out], axis=1)


def kernel(x, emb_table):
    if x.ndim == 3:
        L = x.shape[-2]
        return _pos_add_3d(x, emb_table[:L])
    b, h, l, d = x.shape
    xr = jnp.reshape(jnp.transpose(x, (0, 2, 1, 3)), (b, l, h * d))
    xr = _pos_add_3d(xr, emb_table[:l])
    return jnp.transpose(jnp.reshape(xr, (b, l, h, d)), (0, 2, 1, 3))


# speedup vs baseline: 6.5390x; 3.0706x over previous
import jax
import jax.numpy as jnp
from jax.experimental import pallas as pl


def _add_kernel(x_ref, emb_ref, o_ref):
    o_ref[...] = x_ref[...] + emb_ref[...][None, :, :]


def _pos_add_3d(x, emb_slice):
    B, L, D = x.shape
    BLK = 512
    grid = (L // BLK,)
    return pl.pallas_call(
        _add_kernel,
        grid=grid,
        in_specs=[
            pl.BlockSpec((B, BLK, D), lambda i: (0, i, 0)),
            pl.BlockSpec((BLK, D), lambda i: (i, 0)),
        ],
        out_specs=pl.BlockSpec((B, BLK, D), lambda i: (0, i, 0)),
        out_shape=jax.ShapeDtypeStruct((B, L, D), x.dtype),
    )(x, emb_slice)


def kernel(x, emb_table):
    if x.ndim == 3:
        L = x.shape[-2]
        return _pos_add_3d(x, emb_table[:L])
    b, h, l, d = x.shape
    xr = jnp.reshape(jnp.transpose(x, (0, 2, 1, 3)), (b, l, h * d))
    xr = _pos_add_3d(xr, emb_table[:l])
    return jnp.transpose(jnp.reshape(xr, (b, l, h, d)), (0, 2, 1, 3))
